# baseline (speedup 1.0000x reference)
import functools

import jax
import jax.numpy as jnp
from jax.experimental import pallas as pl
from jax.experimental.pallas import tpu as pltpu

_BM = 200


def _fused_kernel(adj_ref, x_ref, w1_ref, b1_ref, w2_ref, b2_ref, o_ref,
                  xw_s, h1w_s, *, nb, bm):
    i = pl.program_id(0)

    @pl.when(i == 0)
    def _():
        xw_s[...] = jnp.dot(
            x_ref[...], w1_ref[...],
            preferred_element_type=jnp.float32,
            precision=jax.lax.Precision.HIGHEST,
        ).astype(jnp.bfloat16)

    adj_bf = adj_ref[...].astype(jnp.bfloat16)

    @pl.when(i < nb)
    def _():
        acc = jnp.dot(adj_bf, xw_s[...], preferred_element_type=jnp.float32)
        h = acc + b1_ref[...]
        h = jnp.where(h >= 0, h, 0.01 * h)
        h1w_s[pl.ds(i * bm, bm), :] = jnp.dot(
            h, w2_ref[...],
            preferred_element_type=jnp.float32,
            precision=jax.lax.Precision.HIGHEST,
        ).astype(jnp.bfloat16)

    @pl.when(i >= nb)
    def _():
        acc = jnp.dot(adj_bf, h1w_s[...], preferred_element_type=jnp.float32)
        h = acc + b2_ref[...]
        o_ref[...] = jnp.where(h >= 0, h, 0.01 * h)


def kernel(x, adj, W1, b1, W2, b2):
    n, d = adj.shape[0], W1.shape[1]
    nb = n // _BM
    b1r = b1.reshape(1, -1)
    b2r = b2.reshape(1, -1)
    return pl.pallas_call(
        functools.partial(_fused_kernel, nb=nb, bm=_BM),
        grid=(2 * nb,),
        in_specs=[
            pl.BlockSpec((_BM, n), lambda i: (jax.lax.rem(i, nb), 0)),
            pl.BlockSpec(x.shape, lambda i: (0, 0)),
            pl.BlockSpec(W1.shape, lambda i: (0, 0)),
            pl.BlockSpec(b1r.shape, lambda i: (0, 0)),
            pl.BlockSpec(W2.shape, lambda i: (0, 0)),
            pl.BlockSpec(b2r.shape, lambda i: (0, 0)),
        ],
        out_specs=pl.BlockSpec(
            (_BM, d), lambda i: (jnp.maximum(i - nb, 0), 0)
        ),
        out_shape=jax.ShapeDtypeStruct((n, d), jnp.float32),
        scratch_shapes=[
            pltpu.VMEM((n, d), jnp.bfloat16),
            pltpu.VMEM((n, d), jnp.bfloat16),
        ],
    )(adj, x, W1, b1r, W2, b2r)


# speedup vs baseline: 1.0874x; 1.0874x over previous
import functools

import jax
import jax.numpy as jnp
from jax.experimental import pallas as pl
from jax.experimental.pallas import tpu as pltpu

_BM = 400


def _xw_kernel(x_ref, w_ref, o_ref):
    o_ref[...] = jnp.dot(
        x_ref[...], w_ref[...],
        preferred_element_type=jnp.float32,
        precision=jax.lax.Precision.HIGHEST,
    ).astype(jnp.bfloat16)


def _fused_kernel(adj_ref, xw_ref, b1_ref, w2_ref, b2_ref, o_ref,
                  h1w_s, *, nb, bm):
    i = pl.program_id(0)
    adj_bf = adj_ref[...].astype(jnp.bfloat16)

    @pl.when(i < nb)
    def _():
        acc = jnp.dot(adj_bf, xw_ref[...], preferred_element_type=jnp.float32)
        h = acc + b1_ref[...]
        h = jnp.where(h >= 0, h, 0.01 * h)
        h1w_s[pl.ds(i * bm, bm), :] = jnp.dot(
            h, w2_ref[...],
            preferred_element_type=jnp.float32,
            precision=jax.lax.Precision.HIGHEST,
        ).astype(jnp.bfloat16)

    @pl.when(i >= nb)
    def _():
        acc = jnp.dot(adj_bf, h1w_s[...], preferred_element_type=jnp.float32)
        h = acc + b2_ref[...]
        o_ref[...] = jnp.where(h >= 0, h, 0.01 * h)


def kernel(x, adj, W1, b1, W2, b2):
    n, d = adj.shape[0], W1.shape[1]
    nb = n // _BM
    xw = pl.pallas_call(
        _xw_kernel,
        out_shape=jax.ShapeDtypeStruct((n, d), jnp.bfloat16),
    )(x, W1)
    b1r = b1.reshape(1, -1)
    b2r = b2.reshape(1, -1)
    return pl.pallas_call(
        functools.partial(_fused_kernel, nb=nb, bm=_BM),
        grid=(2 * nb,),
        in_specs=[
            pl.BlockSpec((_BM, n), lambda i: (jax.lax.rem(i, nb), 0)),
            pl.BlockSpec(xw.shape, lambda i: (0, 0)),
            pl.BlockSpec(b1r.shape, lambda i: (0, 0)),
            pl.BlockSpec(W2.shape, lambda i: (0, 0)),
            pl.BlockSpec(b2r.shape, lambda i: (0, 0)),
        ],
        out_specs=pl.BlockSpec(
            (_BM, d), lambda i: (jnp.maximum(i - nb, 0), 0)
        ),
        out_shape=jax.ShapeDtypeStruct((n, d), jnp.float32),
        scratch_shapes=[
            pltpu.VMEM((n, d), jnp.bfloat16),
        ],
    )(adj, xw, b1r, W2, b2r)
